# Initial kernel scaffold; baseline (speedup 1.0000x reference)
import jax
import jax.numpy as jnp
from jax.experimental import pallas as pl


def kernel(x, edge_index, precomp, precomp_self, connection, wring1, ang1, wc1, wring_s1, ang_s1, wc_s1, att1, wring2, ang2, wc2, wring_s2, ang_s2, wc_s2, att2):
    raise NotImplementedError("write your pallas kernel here")


# speedup vs baseline: 9.3082x; 9.3082x over previous
import functools
import numpy as np
import jax
import jax.numpy as jnp
from jax import lax
from jax.experimental import pallas as pl
from jax.experimental.pallas import tpu as pltpu
from jax.experimental.pallas import tpu_sc as plsc

N = 10000
E = 160000
C = 64
O = 5
F = 5
R = 6
H = 2
S = 5
LW = 384
HALF = 192
BN = 400
BE = 640
ECHUNK = 200
SCHUNK = 400
GTILES = 32
EPG = E // GTILES
LWH = 128
NHALF = N // 2
ACC_ROWS = 5008
TROWS = 320
TROWS_LAST = ACC_ROWS - 15 * TROWS
TROWS_OUT_LAST = NHALF - 15 * TROWS

_l = np.arange(LW)
_MH0 = ((_l < 320) & (_l % 64 < 32)).astype(np.float32)[None, :]
_MH1 = ((_l < 320) & (_l % 64 >= 32)).astype(np.float32)[None, :]
_D320 = (_l == 320).astype(np.float32)[None, :]
_D321 = (_l == 321).astype(np.float32)[None, :]
def _padlw(row32):
    return np.pad(row32, ((0, 0), (0, LW - 32))).astype(np.float32)

_theta = np.arange(S) * 2.0 * np.pi / S
_B = np.zeros((O, S), dtype=np.float32)
_B[0] = 1.0
for _m in range(1, 3):
    _B[2 * _m - 1] = np.cos(_m * _theta)
    _B[2 * _m] = np.sin(_m * _theta)
_BINV = np.linalg.pinv(_B).astype(np.float32)

_WRMASK = np.zeros((F * R, F), np.float32)
for _f in range(F):
    _WRMASK[_f * R:(_f + 1) * R, _f] = 1.0

_i_of = np.minimum(np.arange(32) // 5, 4)
_M_ONE = (_i_of == 0).astype(np.float32)[None, :] * (np.arange(32) < 25)[None, :]
_M_C1 = (((_i_of == 1) | (_i_of == 2)) & (np.arange(32) < 25)).astype(np.float32)[None, :]
_M_C2 = (((_i_of == 3) | (_i_of == 4)) & (np.arange(32) < 25)).astype(np.float32)[None, :]
_SGN_S1 = np.where(_i_of == 1, 1.0, np.where(_i_of == 2, -1.0, 0.0)).astype(np.float32)[None, :] * (np.arange(32) < 25)[None, :]
_SGN_S2 = np.where(_i_of == 3, 1.0, np.where(_i_of == 4, -1.0, 0.0)).astype(np.float32)[None, :] * (np.arange(32) < 25)[None, :]

_CST = np.concatenate(
    [_MH0, _MH1, _D320, _D321, _padlw(_M_ONE), _padlw(_M_C1), _padlw(_M_C2),
     _padlw(_SGN_S1), _padlw(_SGN_S2), np.zeros((7, LW), np.float32)],
    axis=0).astype(np.float32)


def _prep_body(xin_ref, wc_ref, out_ref):
    wc = wc_ref[...]
    parts = [jnp.dot(xin_ref[:, i * 64:(i + 1) * 64], wc,
                     preferred_element_type=jnp.float32) for i in range(O)]
    parts.append(jnp.zeros((xin_ref.shape[0], LW - 320), jnp.float32))
    out_ref[...] = jnp.concatenate(parts, axis=1)


def _tc_prep(xin, wc):
    return pl.pallas_call(
        _prep_body,
        grid=(N // BN,),
        in_specs=[pl.BlockSpec((BN, LW), lambda i: (i, 0)),
                  pl.BlockSpec((C, C), lambda i: (0, 0))],
        out_specs=pl.BlockSpec((BN, LW), lambda i: (i, 0)),
        out_shape=jax.ShapeDtypeStruct((N, LW), jnp.float32),
    )(xin, wc)


def _edge_body(xs_ref, eaux_ref, wr_ref, angp_ref, attrows_ref, cst_ref,
               t0_ref, t1_ref, t2_ref):
    be = xs_ref.shape[0]
    eaux = eaux_ref[...]
    rad = jnp.dot(eaux, wr_ref[...], preferred_element_type=jnp.float32)
    ker = jnp.dot(rad, angp_ref[...], preferred_element_type=jnp.float32)
    conn = eaux[:, 30:31]
    connb = jnp.broadcast_to(conn, (be, 32))
    cb = jnp.cos(connb)
    sb = jnp.sin(connb)
    c2b = cb * cb - sb * sb
    s2b = 2.0 * cb * sb
    trig_a = (cst_ref[4:5, 0:32] + cst_ref[5:6, 0:32] * cb
              + cst_ref[6:7, 0:32] * c2b)
    trig_b = cst_ref[7:8, 0:32] * sb + cst_ref[8:9, 0:32] * s2b
    zer5 = jnp.zeros((be, 5), jnp.float32)
    zer7 = jnp.zeros((be, 7), jnp.float32)
    kswap = jnp.concatenate([zer5, ker[:, 10:15], ker[:, 5:10],
                             ker[:, 20:25], ker[:, 15:20], zer7], axis=1)
    kerp = ker * trig_a + kswap * trig_b

    xs = xs_ref[...]
    msgs = []
    for o in range(O):
        acc = None
        for i in range(O):
            k = jnp.broadcast_to(kerp[:, i * 5 + o:i * 5 + o + 1], (be, 64))
            t = k * xs[:, i * 64:(i + 1) * 64]
            acc = t if acc is None else acc + t
        msgs.append(acc)
    msgs.append(jnp.zeros((be, LW - 320), jnp.float32))
    msg = jnp.concatenate(msgs, axis=1)

    l0 = jnp.sum(msg * attrows_ref[0:1, :], axis=1, keepdims=True)
    l1 = jnp.sum(msg * attrows_ref[1:2, :], axis=1, keepdims=True)
    l0 = jnp.where(l0 > 0, l0, 0.2 * l0)
    l1 = jnp.where(l1 > 0, l1, 0.2 * l1)
    ex0 = jnp.exp(l0)
    ex1 = jnp.exp(l1)
    scaleb = ex0 * cst_ref[0:1, 0:64] + ex1 * cst_ref[1:2, 0:64]
    wts = [msgs[o] * scaleb for o in range(O)]
    exblk = ex0 * cst_ref[2:3, 320:384] + ex1 * cst_ref[3:4, 320:384]
    t0_ref[...] = jnp.concatenate(wts[0:2], axis=1)
    t1_ref[...] = jnp.concatenate(wts[2:4], axis=1)
    t2_ref[...] = jnp.concatenate([wts[4], exblk], axis=1)


def _tc_edge(xs, eaux, wr, angp, attrows, cst):
    return pl.pallas_call(
        _edge_body,
        grid=(E // BE,),
        in_specs=[pl.BlockSpec((BE, LW), lambda i: (i, 0)),
                  pl.BlockSpec((BE, 32), lambda i: (i, 0)),
                  pl.BlockSpec((32, 8), lambda i: (0, 0)),
                  pl.BlockSpec((8, 32), lambda i: (0, 0)),
                  pl.BlockSpec((2, LW), lambda i: (0, 0)),
                  pl.BlockSpec((16, LW), lambda i: (0, 0))],
        out_specs=[pl.BlockSpec((BE, LWH), lambda i: (i, 0)),
                   pl.BlockSpec((BE, LWH), lambda i: (i, 0)),
                   pl.BlockSpec((BE, LWH), lambda i: (i, 0))],
        out_shape=[jax.ShapeDtypeStruct((E, LWH), jnp.float32),
                   jax.ShapeDtypeStruct((E, LWH), jnp.float32),
                   jax.ShapeDtypeStruct((E, LWH), jnp.float32)],
    )(xs, eaux, wr, angp, attrows, cst)


def _final_body(acc0_ref, acc1_ref, acc2_ref, xin_ref, paux_ref, wcs_ref,
                wrs_ref, angsp_ref, cst_ref, out_ref):
    bn = acc0_ref.shape[0]
    rad_s = jnp.dot(paux_ref[...], wrs_ref[...], preferred_element_type=jnp.float32)
    kers = jnp.dot(rad_s, angsp_ref[...], preferred_element_type=jnp.float32)
    wcs = wcs_ref[...]
    xws = [jnp.dot(xin_ref[:, i * 64:(i + 1) * 64], wcs,
                   preferred_element_type=jnp.float32) for i in range(O)]
    acc = jnp.concatenate([acc0_ref[...], acc1_ref[...], acc2_ref[...]],
                          axis=1)
    den0 = acc[:, 320:321]
    den1 = acc[:, 321:322]
    denb = (den0 * cst_ref[0:1, 0:64] + den1 * cst_ref[1:2, 0:64]) + 1e-9
    inv = 1.0 / denb
    outs = []
    for o in range(O):
        y = acc[:, o * 64:(o + 1) * 64] * inv
        for i in range(O):
            k = jnp.broadcast_to(kers[:, i * 5 + o:i * 5 + o + 1], (bn, 64))
            y = y + k * xws[i]
        outs.append(y)
    outs.append(jnp.zeros((bn, LW - 320), jnp.float32))
    out_ref[...] = jnp.concatenate(outs, axis=1)


def _tc_final(acc0, acc1, acc2, xin, paux, wcs, wrs, angsp, cst):
    return pl.pallas_call(
        _final_body,
        grid=(N // BN,),
        in_specs=[pl.BlockSpec((BN, LWH), lambda i: (i, 0)),
                  pl.BlockSpec((BN, LWH), lambda i: (i, 0)),
                  pl.BlockSpec((BN, LWH), lambda i: (i, 0)),
                  pl.BlockSpec((BN, LW), lambda i: (i, 0)),
                  pl.BlockSpec((BN, 32), lambda i: (i, 0)),
                  pl.BlockSpec((C, C), lambda i: (0, 0)),
                  pl.BlockSpec((32, 8), lambda i: (0, 0)),
                  pl.BlockSpec((8, 32), lambda i: (0, 0)),
                  pl.BlockSpec((16, LW), lambda i: (0, 0))],
        out_specs=pl.BlockSpec((BN, LW), lambda i: (i, 0)),
        out_shape=jax.ShapeDtypeStruct((N, LW), jnp.float32),
    )(acc0, acc1, acc2, xin, paux, wcs, wrs, angsp, cst)


def _finale_body(y_ref, x_ref, out_ref):
    ys = [y_ref[:, o * 64:(o + 1) * 64] + x_ref[:, o * 64:(o + 1) * 64]
          for o in range(O)]
    sigs = []
    for s in range(S):
        a = None
        for o in range(O):
            coef = float(_B[o, s])
            if coef == 0.0:
                continue
            t = coef * ys[o]
            a = t if a is None else a + t
        a = jnp.maximum(a, 0.0)
        sigs.append(a)
    outs = []
    for o in range(O):
        a = None
        for s in range(S):
            coef = float(_BINV[s, o])
            t = coef * sigs[s]
            a = t if a is None else a + t
        outs.append(a)
    out_ref[...] = jnp.concatenate(outs, axis=1)


def _tc_finale(y, xpad):
    return pl.pallas_call(
        _finale_body,
        grid=(N // BN,),
        in_specs=[pl.BlockSpec((BN, LW), lambda i: (i, 0)),
                  pl.BlockSpec((BN, LW), lambda i: (i, 0))],
        out_specs=pl.BlockSpec((BN, 320), lambda i: (i, 0)),
        out_shape=jax.ShapeDtypeStruct((N, 320), jnp.float32),
    )(y, xpad)


def _gather_body(table_hbm, idx_hbm, out_hbm, idx_v, rows_v):
    wid = lax.axis_index("s") * 2 + lax.axis_index("c")
    base = wid * EPG

    def chunk(cc, _):
        e0 = base + cc * ECHUNK
        pltpu.sync_copy(idx_hbm.at[pl.ds(e0, ECHUNK)], idx_v)
        pltpu.sync_copy(table_hbm.at[idx_v], rows_v)
        pltpu.sync_copy(rows_v, out_hbm.at[pl.ds(e0, ECHUNK)])
        return 0

    lax.fori_loop(0, EPG // ECHUNK, chunk, 0)


@functools.lru_cache(maxsize=None)
def _build_gather():
    mesh = plsc.VectorSubcoreMesh(core_axis_name="c", subcore_axis_name="s")
    return pl.kernel(
        _gather_body,
        out_type=jax.ShapeDtypeStruct((E, LW), jnp.float32),
        mesh=mesh,
        scratch_types=[pltpu.VMEM((ECHUNK,), jnp.int32),
                       pltpu.VMEM((ECHUNK, LW), jnp.float32)],
    )


def _sc_gather(table, idx):
    return _build_gather()(table, idx)


def _scatter_body(wt_hbm, idx_hbm, zeros_hbm, out_hbm, idx_v, buf_v, acc_sp):
    c = lax.axis_index("c")
    s = lax.axis_index("s")
    node0 = c * NHALF
    row0 = s * TROWS

    pltpu.sync_copy(zeros_hbm, buf_v.at[pl.ds(0, TROWS)])

    @pl.when(s == 15)
    def _():
        pltpu.sync_copy(buf_v.at[pl.ds(0, TROWS_LAST)],
                        acc_sp.at[pl.ds(row0, TROWS_LAST)])

    @pl.when(s != 15)
    def _():
        pltpu.sync_copy(buf_v.at[pl.ds(0, TROWS)], acc_sp.at[pl.ds(row0, TROWS)])

    plsc.subcore_barrier()

    def chunk(cc, _):
        e0 = s * (E // 16) + cc * SCHUNK
        pltpu.sync_copy(idx_hbm.at[pl.ds(e0, SCHUNK)], idx_v)
        for j in range(SCHUNK // 16):
            v = idx_v[pl.ds(j * 16, 16)] - node0
            ok = (v >= 0) & (v < NHALF)
            idx_v[pl.ds(j * 16, 16)] = jnp.where(ok, v, NHALF)
        pltpu.sync_copy(wt_hbm.at[pl.ds(e0, SCHUNK)], buf_v.at[pl.ds(0, SCHUNK)])
        pltpu.sync_copy(buf_v.at[pl.ds(0, SCHUNK)], acc_sp.at[idx_v], add=True)
        return 0

    lax.fori_loop(0, (E // 16) // SCHUNK, chunk, 0)
    plsc.subcore_barrier()

    @pl.when(s == 15)
    def _():
        pltpu.sync_copy(acc_sp.at[pl.ds(row0, TROWS_OUT_LAST)],
                        buf_v.at[pl.ds(0, TROWS_OUT_LAST)])
        pltpu.sync_copy(buf_v.at[pl.ds(0, TROWS_OUT_LAST)],
                        out_hbm.at[pl.ds(node0 + row0, TROWS_OUT_LAST)])

    @pl.when(s != 15)
    def _():
        pltpu.sync_copy(acc_sp.at[pl.ds(row0, TROWS)], buf_v.at[pl.ds(0, TROWS)])
        pltpu.sync_copy(buf_v.at[pl.ds(0, TROWS)],
                        out_hbm.at[pl.ds(node0 + row0, TROWS)])


@functools.lru_cache(maxsize=None)
def _build_scatter():
    mesh = plsc.VectorSubcoreMesh(core_axis_name="c", subcore_axis_name="s")
    return pl.kernel(
        _scatter_body,
        out_type=jax.ShapeDtypeStruct((N, LWH), jnp.float32),
        mesh=mesh,
        scratch_types=[pltpu.VMEM((SCHUNK,), jnp.int32),
                       pltpu.VMEM((SCHUNK, LWH), jnp.float32),
                       pltpu.VMEM_SHARED((ACC_ROWS, LWH), jnp.float32)],
    )


def _sc_scatter(wt, dst, zeros_init):
    return _build_scatter()(wt, dst, zeros_init)


def _prep_weights(wring, ang, att):
    wr = jnp.tile(wring, F)[:, None] * jnp.asarray(_WRMASK)
    wr = jnp.pad(wr, ((0, 2), (0, 3)))
    angp = jnp.pad(ang.reshape(F, O * O), ((0, 3), (0, 7)))
    a0 = att[0].T
    a1 = att[1].T
    row = jnp.concatenate([a0, a1], axis=1).reshape(1, 320)
    row = jnp.pad(row, ((0, 0), (0, LW - 320)))
    attrows = jnp.concatenate([row * jnp.asarray(_MH0),
                               row * jnp.asarray(_MH1)], axis=0)
    return wr, angp, attrows


def kernel(x, edge_index, precomp, precomp_self, connection,
           wring1, ang1, wc1, wring_s1, ang_s1, wc_s1, att1,
           wring2, ang2, wc2, wring_s2, ang_s2, wc_s2, att2):
    src = edge_index[:, 0]
    dst = edge_index[:, 1]
    xpad = jnp.pad(x.transpose(0, 2, 1).reshape(N, 320),
                   ((0, 0), (0, LW - 320)))
    eaux = jnp.concatenate([precomp.reshape(E, F * R), connection[:, None],
                            jnp.zeros((E, 1), jnp.float32)], axis=1)
    paux = jnp.pad(precomp_self.reshape(N, F * R), ((0, 0), (0, 2)))
    cst = jnp.asarray(_CST)
    zeros_init = jnp.zeros((TROWS, LWH), jnp.float32)

    wr1, angp1, attrows1 = _prep_weights(wring1, ang1, att1)
    wr2, angp2, attrows2 = _prep_weights(wring2, ang2, att2)
    wrs1, angsp1, _ = _prep_weights(wring_s1, ang_s1, att1)
    wrs2, angsp2, _ = _prep_weights(wring_s2, ang_s2, att2)

    xw1 = _tc_prep(xpad, wc1)
    xs1 = _sc_gather(xw1, src)
    wt1 = _tc_edge(xs1, eaux, wr1, angp1, attrows1, cst)
    accs1 = [_sc_scatter(w, dst, zeros_init) for w in wt1]
    y1 = _tc_final(*accs1, xpad, paux, wc_s1, wrs1, angsp1, cst)

    xw2 = _tc_prep(y1, wc2)
    xs2 = _sc_gather(xw2, src)
    wt2 = _tc_edge(xs2, eaux, wr2, angp2, attrows2, cst)
    accs2 = [_sc_scatter(w, dst, zeros_init) for w in wt2]
    y2 = _tc_final(*accs2, y1, paux, wc_s2, wrs2, angsp2, cst)

    out = _tc_finale(y2, xpad)
    return out.reshape(N, O, C).transpose(0, 2, 1)


# speedup vs baseline: 9.3262x; 1.0019x over previous
import functools
import numpy as np
import jax
import jax.numpy as jnp
from jax import lax
from jax.experimental import pallas as pl
from jax.experimental.pallas import tpu as pltpu
from jax.experimental.pallas import tpu_sc as plsc

N = 10000
E = 160000
C = 64
O = 5
F = 5
R = 6
H = 2
S = 5
LW = 384
HALF = 192
BN = 400
BE = 640
ECHUNK = 200
SCHUNK = 80
GTILES = 32
EPG = E // GTILES
LWH = 128
NHALF = N // 2
ACC_ROWS = 5008
TROWS = 320
TROWS_LAST = ACC_ROWS - 15 * TROWS
TROWS_OUT_LAST = NHALF - 15 * TROWS

_l = np.arange(LW)
_MH0 = ((_l < 320) & (_l % 64 < 32)).astype(np.float32)[None, :]
_MH1 = ((_l < 320) & (_l % 64 >= 32)).astype(np.float32)[None, :]
_D320 = (_l == 320).astype(np.float32)[None, :]
_D321 = (_l == 321).astype(np.float32)[None, :]
def _padlw(row32):
    return np.pad(row32, ((0, 0), (0, LW - 32))).astype(np.float32)

_theta = np.arange(S) * 2.0 * np.pi / S
_B = np.zeros((O, S), dtype=np.float32)
_B[0] = 1.0
for _m in range(1, 3):
    _B[2 * _m - 1] = np.cos(_m * _theta)
    _B[2 * _m] = np.sin(_m * _theta)
_BINV = np.linalg.pinv(_B).astype(np.float32)

_WRMASK = np.zeros((F * R, F), np.float32)
for _f in range(F):
    _WRMASK[_f * R:(_f + 1) * R, _f] = 1.0

_i_of = np.minimum(np.arange(32) // 5, 4)
_M_ONE = (_i_of == 0).astype(np.float32)[None, :] * (np.arange(32) < 25)[None, :]
_M_C1 = (((_i_of == 1) | (_i_of == 2)) & (np.arange(32) < 25)).astype(np.float32)[None, :]
_M_C2 = (((_i_of == 3) | (_i_of == 4)) & (np.arange(32) < 25)).astype(np.float32)[None, :]
_SGN_S1 = np.where(_i_of == 1, 1.0, np.where(_i_of == 2, -1.0, 0.0)).astype(np.float32)[None, :] * (np.arange(32) < 25)[None, :]
_SGN_S2 = np.where(_i_of == 3, 1.0, np.where(_i_of == 4, -1.0, 0.0)).astype(np.float32)[None, :] * (np.arange(32) < 25)[None, :]

_CST = np.concatenate(
    [_MH0, _MH1, _D320, _D321, _padlw(_M_ONE), _padlw(_M_C1), _padlw(_M_C2),
     _padlw(_SGN_S1), _padlw(_SGN_S2), np.zeros((7, LW), np.float32)],
    axis=0).astype(np.float32)


def _prep_body(xin_ref, wc_ref, out_ref):
    wc = wc_ref[...]
    parts = [jnp.dot(xin_ref[:, i * 64:(i + 1) * 64], wc,
                     preferred_element_type=jnp.float32) for i in range(O)]
    parts.append(jnp.zeros((xin_ref.shape[0], LW - 320), jnp.float32))
    out_ref[...] = jnp.concatenate(parts, axis=1)


def _tc_prep(xin, wc):
    return pl.pallas_call(
        _prep_body,
        grid=(N // BN,),
        in_specs=[pl.BlockSpec((BN, LW), lambda i: (i, 0)),
                  pl.BlockSpec((C, C), lambda i: (0, 0))],
        out_specs=pl.BlockSpec((BN, LW), lambda i: (i, 0)),
        out_shape=jax.ShapeDtypeStruct((N, LW), jnp.float32),
    )(xin, wc)


def _edge_body(xs_ref, eaux_ref, wr_ref, angp_ref, attrows_ref, cst_ref,
               t0_ref, t1_ref, t2_ref):
    be = xs_ref.shape[0]
    eaux = eaux_ref[...]
    rad = jnp.dot(eaux, wr_ref[...], preferred_element_type=jnp.float32)
    ker = jnp.dot(rad, angp_ref[...], preferred_element_type=jnp.float32)
    conn = eaux[:, 30:31]
    connb = jnp.broadcast_to(conn, (be, 32))
    cb = jnp.cos(connb)
    sb = jnp.sin(connb)
    c2b = cb * cb - sb * sb
    s2b = 2.0 * cb * sb
    trig_a = (cst_ref[4:5, 0:32] + cst_ref[5:6, 0:32] * cb
              + cst_ref[6:7, 0:32] * c2b)
    trig_b = cst_ref[7:8, 0:32] * sb + cst_ref[8:9, 0:32] * s2b
    zer5 = jnp.zeros((be, 5), jnp.float32)
    zer7 = jnp.zeros((be, 7), jnp.float32)
    kswap = jnp.concatenate([zer5, ker[:, 10:15], ker[:, 5:10],
                             ker[:, 20:25], ker[:, 15:20], zer7], axis=1)
    kerp = ker * trig_a + kswap * trig_b

    xs = xs_ref[...]
    msgs = []
    for o in range(O):
        acc = None
        for i in range(O):
            k = jnp.broadcast_to(kerp[:, i * 5 + o:i * 5 + o + 1], (be, 64))
            t = k * xs[:, i * 64:(i + 1) * 64]
            acc = t if acc is None else acc + t
        msgs.append(acc)
    msgs.append(jnp.zeros((be, LW - 320), jnp.float32))
    msg = jnp.concatenate(msgs, axis=1)

    l0 = jnp.sum(msg * attrows_ref[0:1, :], axis=1, keepdims=True)
    l1 = jnp.sum(msg * attrows_ref[1:2, :], axis=1, keepdims=True)
    l0 = jnp.where(l0 > 0, l0, 0.2 * l0)
    l1 = jnp.where(l1 > 0, l1, 0.2 * l1)
    ex0 = jnp.exp(l0)
    ex1 = jnp.exp(l1)
    scaleb = ex0 * cst_ref[0:1, 0:64] + ex1 * cst_ref[1:2, 0:64]
    wts = [msgs[o] * scaleb for o in range(O)]
    exblk = ex0 * cst_ref[2:3, 320:384] + ex1 * cst_ref[3:4, 320:384]
    t0_ref[...] = jnp.concatenate(wts[0:2], axis=1)
    t1_ref[...] = jnp.concatenate(wts[2:4], axis=1)
    t2_ref[...] = jnp.concatenate([wts[4], exblk], axis=1)


def _tc_edge(xs, eaux, wr, angp, attrows, cst):
    return pl.pallas_call(
        _edge_body,
        grid=(E // BE,),
        in_specs=[pl.BlockSpec((BE, LW), lambda i: (i, 0)),
                  pl.BlockSpec((BE, 32), lambda i: (i, 0)),
                  pl.BlockSpec((32, 8), lambda i: (0, 0)),
                  pl.BlockSpec((8, 32), lambda i: (0, 0)),
                  pl.BlockSpec((2, LW), lambda i: (0, 0)),
                  pl.BlockSpec((16, LW), lambda i: (0, 0))],
        out_specs=[pl.BlockSpec((BE, LWH), lambda i: (i, 0)),
                   pl.BlockSpec((BE, LWH), lambda i: (i, 0)),
                   pl.BlockSpec((BE, LWH), lambda i: (i, 0))],
        out_shape=[jax.ShapeDtypeStruct((E, LWH), jnp.float32),
                   jax.ShapeDtypeStruct((E, LWH), jnp.float32),
                   jax.ShapeDtypeStruct((E, LWH), jnp.float32)],
    )(xs, eaux, wr, angp, attrows, cst)


def _final_body(acc0_ref, acc1_ref, acc2_ref, xin_ref, paux_ref, wcs_ref,
                wrs_ref, angsp_ref, cst_ref, out_ref):
    bn = acc0_ref.shape[0]
    rad_s = jnp.dot(paux_ref[...], wrs_ref[...], preferred_element_type=jnp.float32)
    kers = jnp.dot(rad_s, angsp_ref[...], preferred_element_type=jnp.float32)
    wcs = wcs_ref[...]
    xws = [jnp.dot(xin_ref[:, i * 64:(i + 1) * 64], wcs,
                   preferred_element_type=jnp.float32) for i in range(O)]
    acc = jnp.concatenate([acc0_ref[...], acc1_ref[...], acc2_ref[...]],
                          axis=1)
    den0 = acc[:, 320:321]
    den1 = acc[:, 321:322]
    denb = (den0 * cst_ref[0:1, 0:64] + den1 * cst_ref[1:2, 0:64]) + 1e-9
    inv = 1.0 / denb
    outs = []
    for o in range(O):
        y = acc[:, o * 64:(o + 1) * 64] * inv
        for i in range(O):
            k = jnp.broadcast_to(kers[:, i * 5 + o:i * 5 + o + 1], (bn, 64))
            y = y + k * xws[i]
        outs.append(y)
    outs.append(jnp.zeros((bn, LW - 320), jnp.float32))
    out_ref[...] = jnp.concatenate(outs, axis=1)


def _tc_final(acc0, acc1, acc2, xin, paux, wcs, wrs, angsp, cst):
    return pl.pallas_call(
        _final_body,
        grid=(N // BN,),
        in_specs=[pl.BlockSpec((BN, LWH), lambda i: (i, 0)),
                  pl.BlockSpec((BN, LWH), lambda i: (i, 0)),
                  pl.BlockSpec((BN, LWH), lambda i: (i, 0)),
                  pl.BlockSpec((BN, LW), lambda i: (i, 0)),
                  pl.BlockSpec((BN, 32), lambda i: (i, 0)),
                  pl.BlockSpec((C, C), lambda i: (0, 0)),
                  pl.BlockSpec((32, 8), lambda i: (0, 0)),
                  pl.BlockSpec((8, 32), lambda i: (0, 0)),
                  pl.BlockSpec((16, LW), lambda i: (0, 0))],
        out_specs=pl.BlockSpec((BN, LW), lambda i: (i, 0)),
        out_shape=jax.ShapeDtypeStruct((N, LW), jnp.float32),
    )(acc0, acc1, acc2, xin, paux, wcs, wrs, angsp, cst)


def _finale_body(y_ref, x_ref, out_ref):
    ys = [y_ref[:, o * 64:(o + 1) * 64] + x_ref[:, o * 64:(o + 1) * 64]
          for o in range(O)]
    sigs = []
    for s in range(S):
        a = None
        for o in range(O):
            coef = float(_B[o, s])
            if coef == 0.0:
                continue
            t = coef * ys[o]
            a = t if a is None else a + t
        a = jnp.maximum(a, 0.0)
        sigs.append(a)
    outs = []
    for o in range(O):
        a = None
        for s in range(S):
            coef = float(_BINV[s, o])
            t = coef * sigs[s]
            a = t if a is None else a + t
        outs.append(a)
    out_ref[...] = jnp.concatenate(outs, axis=1)


def _tc_finale(y, xpad):
    return pl.pallas_call(
        _finale_body,
        grid=(N // BN,),
        in_specs=[pl.BlockSpec((BN, LW), lambda i: (i, 0)),
                  pl.BlockSpec((BN, LW), lambda i: (i, 0))],
        out_specs=pl.BlockSpec((BN, 320), lambda i: (i, 0)),
        out_shape=jax.ShapeDtypeStruct((N, 320), jnp.float32),
    )(y, xpad)


def _gather_body(table_hbm, idx_hbm, out_hbm, idx_v, rows_v):
    wid = lax.axis_index("s") * 2 + lax.axis_index("c")
    base = wid * EPG

    def chunk(cc, _):
        e0 = base + cc * ECHUNK
        pltpu.sync_copy(idx_hbm.at[pl.ds(e0, ECHUNK)], idx_v)
        pltpu.sync_copy(table_hbm.at[idx_v], rows_v)
        pltpu.sync_copy(rows_v, out_hbm.at[pl.ds(e0, ECHUNK)])
        return 0

    lax.fori_loop(0, EPG // ECHUNK, chunk, 0)


@functools.lru_cache(maxsize=None)
def _build_gather():
    mesh = plsc.VectorSubcoreMesh(core_axis_name="c", subcore_axis_name="s")
    return pl.kernel(
        _gather_body,
        out_type=jax.ShapeDtypeStruct((E, LW), jnp.float32),
        mesh=mesh,
        scratch_types=[pltpu.VMEM((ECHUNK,), jnp.int32),
                       pltpu.VMEM((ECHUNK, LW), jnp.float32)],
    )


def _sc_gather(table, idx):
    return _build_gather()(table, idx)


def _remap(idx_v, node0):
    for j in range(SCHUNK // 16):
        v = idx_v[pl.ds(j * 16, 16)] - node0
        ok = (v >= 0) & (v < NHALF)
        idx_v[pl.ds(j * 16, 16)] = jnp.where(ok, v, NHALF)


def _scatter_body(wt_hbm, idx_hbm, zeros_hbm, out_hbm,
                  idxa_v, idxb_v, bufa_v, bufb_v, bufs_v, acc_sp, sema, semb):
    c = lax.axis_index("c")
    s = lax.axis_index("s")
    node0 = c * NHALF
    row0 = s * TROWS
    npair = ((E // 16) // SCHUNK) // 2

    pltpu.sync_copy(zeros_hbm, bufs_v)

    @pl.when(s == 15)
    def _():
        pltpu.sync_copy(bufs_v.at[pl.ds(0, TROWS_LAST)],
                        acc_sp.at[pl.ds(row0, TROWS_LAST)])

    @pl.when(s != 15)
    def _():
        pltpu.sync_copy(bufs_v, acc_sp.at[pl.ds(row0, TROWS)])

    plsc.subcore_barrier()

    def pair(cc, _):
        ea = s * (E // 16) + (2 * cc) * SCHUNK
        eb = ea + SCHUNK
        pltpu.sync_copy(idx_hbm.at[pl.ds(ea, SCHUNK)], idxa_v)
        ha = pltpu.async_copy(wt_hbm.at[pl.ds(ea, SCHUNK)], bufa_v, sema)
        pltpu.sync_copy(idx_hbm.at[pl.ds(eb, SCHUNK)], idxb_v)
        hb = pltpu.async_copy(wt_hbm.at[pl.ds(eb, SCHUNK)], bufb_v, semb)
        _remap(idxa_v, node0)
        _remap(idxb_v, node0)
        ha.wait()
        pltpu.sync_copy(bufa_v, acc_sp.at[idxa_v], add=True)
        hb.wait()
        pltpu.sync_copy(bufb_v, acc_sp.at[idxb_v], add=True)
        return 0

    lax.fori_loop(0, npair, pair, 0)
    et = s * (E // 16) + (2 * npair) * SCHUNK
    pltpu.sync_copy(idx_hbm.at[pl.ds(et, SCHUNK)], idxa_v)
    pltpu.sync_copy(wt_hbm.at[pl.ds(et, SCHUNK)], bufa_v)
    _remap(idxa_v, node0)
    pltpu.sync_copy(bufa_v, acc_sp.at[idxa_v], add=True)

    plsc.subcore_barrier()

    @pl.when(s == 15)
    def _():
        pltpu.sync_copy(acc_sp.at[pl.ds(row0, TROWS_OUT_LAST)],
                        bufs_v.at[pl.ds(0, TROWS_OUT_LAST)])
        pltpu.sync_copy(bufs_v.at[pl.ds(0, TROWS_OUT_LAST)],
                        out_hbm.at[pl.ds(node0 + row0, TROWS_OUT_LAST)])

    @pl.when(s != 15)
    def _():
        pltpu.sync_copy(acc_sp.at[pl.ds(row0, TROWS)], bufs_v)
        pltpu.sync_copy(bufs_v, out_hbm.at[pl.ds(node0 + row0, TROWS)])


@functools.lru_cache(maxsize=None)
def _build_scatter():
    mesh = plsc.VectorSubcoreMesh(core_axis_name="c", subcore_axis_name="s")
    return pl.kernel(
        _scatter_body,
        out_type=jax.ShapeDtypeStruct((N, LWH), jnp.float32),
        mesh=mesh,
        scratch_types=[pltpu.VMEM((SCHUNK,), jnp.int32),
                       pltpu.VMEM((SCHUNK,), jnp.int32),
                       pltpu.VMEM((SCHUNK, LWH), jnp.float32),
                       pltpu.VMEM((SCHUNK, LWH), jnp.float32),
                       pltpu.VMEM((TROWS, LWH), jnp.float32),
                       pltpu.VMEM_SHARED((ACC_ROWS, LWH), jnp.float32),
                       pltpu.SemaphoreType.DMA,
                       pltpu.SemaphoreType.DMA],
    )


def _sc_scatter(wt, dst, zeros_init):
    return _build_scatter()(wt, dst, zeros_init)


def _prep_weights(wring, ang, att):
    wr = jnp.tile(wring, F)[:, None] * jnp.asarray(_WRMASK)
    wr = jnp.pad(wr, ((0, 2), (0, 3)))
    angp = jnp.pad(ang.reshape(F, O * O), ((0, 3), (0, 7)))
    a0 = att[0].T
    a1 = att[1].T
    row = jnp.concatenate([a0, a1], axis=1).reshape(1, 320)
    row = jnp.pad(row, ((0, 0), (0, LW - 320)))
    attrows = jnp.concatenate([row * jnp.asarray(_MH0),
                               row * jnp.asarray(_MH1)], axis=0)
    return wr, angp, attrows


def kernel(x, edge_index, precomp, precomp_self, connection,
           wring1, ang1, wc1, wring_s1, ang_s1, wc_s1, att1,
           wring2, ang2, wc2, wring_s2, ang_s2, wc_s2, att2):
    src = edge_index[:, 0]
    dst = edge_index[:, 1]
    xpad = jnp.pad(x.transpose(0, 2, 1).reshape(N, 320),
                   ((0, 0), (0, LW - 320)))
    eaux = jnp.concatenate([precomp.reshape(E, F * R), connection[:, None],
                            jnp.zeros((E, 1), jnp.float32)], axis=1)
    paux = jnp.pad(precomp_self.reshape(N, F * R), ((0, 0), (0, 2)))
    cst = jnp.asarray(_CST)
    zeros_init = jnp.zeros((TROWS, LWH), jnp.float32)

    wr1, angp1, attrows1 = _prep_weights(wring1, ang1, att1)
    wr2, angp2, attrows2 = _prep_weights(wring2, ang2, att2)
    wrs1, angsp1, _ = _prep_weights(wring_s1, ang_s1, att1)
    wrs2, angsp2, _ = _prep_weights(wring_s2, ang_s2, att2)

    xw1 = _tc_prep(xpad, wc1)
    xs1 = _sc_gather(xw1, src)
    wt1 = _tc_edge(xs1, eaux, wr1, angp1, attrows1, cst)
    accs1 = [_sc_scatter(w, dst, zeros_init) for w in wt1]
    y1 = _tc_final(*accs1, xpad, paux, wc_s1, wrs1, angsp1, cst)

    xw2 = _tc_prep(y1, wc2)
    xs2 = _sc_gather(xw2, src)
    wt2 = _tc_edge(xs2, eaux, wr2, angp2, attrows2, cst)
    accs2 = [_sc_scatter(w, dst, zeros_init) for w in wt2]
    y2 = _tc_final(*accs2, y1, paux, wc_s2, wrs2, angsp2, cst)

    out = _tc_finale(y2, xpad)
    return out.reshape(N, O, C).transpose(0, 2, 1)
